# baseline (speedup 1.0000x reference)
import jax
import jax.numpy as jnp
from jax.experimental import pallas as pl
from jax.experimental.pallas import tpu as pltpu

_H = 256
_W = 256
_TANFOV = 0.7
_MIN_DEPTH = 0.2
_ALPHA_MIN = 1.0 / 255.0
_FX = _W / (2.0 * _TANFOV)
_FY = _H / (2.0 * _TANFOV)
_P = 2048
_G = 128


def _b(v):
    return v.astype(jnp.bfloat16).astype(jnp.float32)


def _prepass_body(g_ref, vm_ref, par_ref, rad_ref):
    m0 = g_ref[0:1, :]
    m1 = g_ref[1:2, :]
    m2 = g_ref[2:3, :]
    s0 = g_ref[3:4, :]
    s1 = g_ref[4:5, :]
    s2 = g_ref[5:6, :]
    qr = g_ref[6:7, :]
    qx = g_ref[7:8, :]
    qy = g_ref[8:9, :]
    qz = g_ref[9:10, :]
    opa = g_ref[10:11, :]

    vm = [[vm_ref[j, k] for k in range(4)] for j in range(4)]
    rb = [[_b(vm[j][k]) for k in range(3)] for j in range(3)]
    mb0, mb1, mb2 = _b(m0), _b(m1), _b(m2)
    xv = ((mb0 * rb[0][0] + mb1 * rb[0][1]) + mb2 * rb[0][2]) + vm[0][3]
    yv = ((mb0 * rb[1][0] + mb1 * rb[1][1]) + mb2 * rb[1][2]) + vm[1][3]
    zv = ((mb0 * rb[2][0] + mb1 * rb[2][1]) + mb2 * rb[2][2]) + vm[2][3]

    zc = jnp.maximum(zv, 1e-4)
    lim_x = 1.3 * _TANFOV
    lim_y = 1.3 * _TANFOV
    txc = jnp.clip(xv / zc, -lim_x, lim_x) * zc
    tyc = jnp.clip(yv / zc, -lim_y, lim_y) * zc

    qnorm = jnp.sqrt(qr * qr + qx * qx + qy * qy + qz * qz)
    r_ = qr / qnorm
    x_ = qx / qnorm
    y_ = qy / qnorm
    z_ = qz / qnorm
    r00 = 1.0 - 2.0 * (y_ * y_ + z_ * z_)
    r01 = 2.0 * (x_ * y_ - r_ * z_)
    r02 = 2.0 * (x_ * z_ + r_ * y_)
    r10 = 2.0 * (x_ * y_ + r_ * z_)
    r11 = 1.0 - 2.0 * (x_ * x_ + z_ * z_)
    r12 = 2.0 * (y_ * z_ - r_ * x_)
    r20 = 2.0 * (x_ * z_ - r_ * y_)
    r21 = 2.0 * (y_ * z_ + r_ * x_)
    r22 = 1.0 - 2.0 * (x_ * x_ + y_ * y_)

    mb = [[_b(r00 * s0), _b(r01 * s1), _b(r02 * s2)],
          [_b(r10 * s0), _b(r11 * s1), _b(r12 * s2)],
          [_b(r20 * s0), _b(r21 * s1), _b(r22 * s2)]]
    sg = [[None] * 3 for _ in range(3)]
    for si in range(3):
        for sj in range(si, 3):
            sg[si][sj] = ((mb[si][0] * mb[sj][0] + mb[si][1] * mb[sj][1])
                          + mb[si][2] * mb[sj][2])
            sg[sj][si] = sg[si][sj]

    j00 = _b(_FX / zc)
    j02 = _b(-_FX * txc / (zc * zc))
    j11 = _b(_FY / zc)
    j12 = _b(-_FY * tyc / (zc * zc))
    t = [[j00 * rb[0][k] + j02 * rb[2][k] for k in range(3)],
         [j11 * rb[1][k] + j12 * rb[2][k] for k in range(3)]]

    tb = [[_b(t[ti][k]) for k in range(3)] for ti in range(2)]
    sgb = [[_b(sg[si][sj]) for sj in range(3)] for si in range(3)]
    u = [[((tb[ti][0] * sgb[0][k] + tb[ti][1] * sgb[1][k])
           + tb[ti][2] * sgb[2][k]) for k in range(3)] for ti in range(2)]
    ub = [[_b(u[ti][k]) for k in range(3)] for ti in range(2)]
    a = ((ub[0][0] * tb[0][0] + ub[0][1] * tb[0][1]) + ub[0][2] * tb[0][2]) + 0.3
    b = ((ub[0][0] * tb[1][0] + ub[0][1] * tb[1][1]) + ub[0][2] * tb[1][2])
    c = ((ub[1][0] * tb[1][0] + ub[1][1] * tb[1][1]) + ub[1][2] * tb[1][2]) + 0.3

    det = a * c - b * b
    inv_det = 1.0 / jnp.maximum(det, 1e-12)
    ca = c * inv_det
    cb = -b * inv_det
    cc = a * inv_det
    mid = 0.5 * (a + c)
    lam1 = mid + jnp.sqrt(jnp.maximum(mid * mid - det, 0.1))
    radii = jnp.ceil(3.0 * jnp.sqrt(lam1)).astype(jnp.int32)
    mx = _FX * xv / zc + 0.5 * _W - 0.5
    my = _FY * yv / zc + 0.5 * _H - 0.5
    validf = jnp.where((zv > _MIN_DEPTH) & (det > 0.0), 1.0, 0.0)

    opa_eff = opa * validf

    amin_s = _ALPHA_MIN * (1.0 - 1e-6)
    qmax = 2.0 * jnp.log(jnp.maximum(opa_eff, 1e-30) / amin_s)
    ry = jnp.where(qmax > 0.0,
                   jnp.sqrt(jnp.maximum(qmax, 0.0) * c) * (1.0 + 1e-5) + 0.01,
                   -1e9)
    rx = jnp.sqrt(jnp.maximum(qmax, 0.0) * a) * (1.0 + 1e-5) + 0.01
    x_on = (mx + rx >= -0.01) & (mx - rx <= (_W - 1) + 0.01)
    reach = jnp.where(x_on, ry, -1e9)

    par_ref[0:1, :] = mx
    par_ref[1:2, :] = my
    par_ref[2:3, :] = ca
    par_ref[3:4, :] = cb
    par_ref[4:5, :] = cc
    par_ref[5:6, :] = opa_eff
    par_ref[6:7, :] = reach
    par_ref[7:8, :] = zv
    par_ref[8:11, :] = g_ref[11:14, :]
    par_ref[11:12, :] = zv
    par_ref[12:16, :] = jnp.zeros((4, zv.shape[1]), jnp.float32)
    rad_ref[...] = radii


def _split3(v):
    vh = v.astype(jnp.bfloat16)
    v1 = v - vh.astype(jnp.float32)
    vm = v1.astype(jnp.bfloat16)
    vl = (v1 - vm.astype(jnp.float32)).astype(jnp.bfloat16)
    return vh, vm, vl


def _sortbin_body(par_ref, parT_ref, comb_ref, posm_ref):
    n = par_ref.shape[1]
    nt = posm_ref.shape[1]
    rpt = _P // _W
    f32 = jnp.float32
    io0 = jax.lax.broadcasted_iota(jnp.int32, (n, n), 0)
    io1 = jax.lax.broadcasted_iota(jnp.int32, (n, n), 1)
    dn_row = (((1,), (0,)), ((), ()))

    zr = par_ref[7:8, :]
    zc = parT_ref[:, 7:8]
    mm = (zc < zr) | ((zc == zr) & (io0 < io1))
    mf = jnp.where(mm, 1.0, 0.0).astype(jnp.bfloat16)
    ones_row = jnp.ones((1, n), jnp.bfloat16)
    rank = jax.lax.dot_general(ones_row, mf, dn_row,
                               preferred_element_type=f32)
    ranki = rank.astype(jnp.int32)
    psort = jnp.where(jnp.broadcast_to(ranki, (n, n)) == io0,
                      1.0, 0.0).astype(jnp.bfloat16)

    parT = parT_ref[...]
    ph, pm, plo = _split3(parT)
    comb_s = (jax.lax.dot_general(psort, ph, dn_row,
                                  preferred_element_type=f32)
              + jax.lax.dot_general(psort, pm, dn_row,
                                    preferred_element_type=f32)
              + jax.lax.dot_general(psort, plo, dn_row,
                                    preferred_element_type=f32))
    comb_ref[...] = comb_s

    my_sc = comb_s[:, 1:2]
    re_sc = comb_s[:, 6:7]
    y0 = (jax.lax.broadcasted_iota(jnp.int32, (n, nt), 1)
          * rpt).astype(f32)
    myb = jnp.broadcast_to(my_sc, (n, nt))
    reb = jnp.broadcast_to(re_sc, (n, nt))
    touch = (myb + reb >= y0) & (myb - reb <= y0 + (rpt - 1))
    touch16 = jnp.where(touch, 1.0, 0.0).astype(jnp.bfloat16)
    low_incl = jnp.where(io1 <= io0, 1.0, 0.0).astype(jnp.bfloat16)
    pos = jax.lax.dot_general(low_incl, touch16, dn_row,
                              preferred_element_type=f32)
    posm_ref[...] = jnp.where(touch, pos - 1.0, -1.0).astype(jnp.int32)


def _raster_body(comb_ref, pos_ref, bg_ref,
                 pix_ref, obs_ref, acc_s, tc_s, wmax_s, obs_s):
    i = pl.program_id(0)
    base = i * _P
    n = comb_ref.shape[0]
    pidx = jax.lax.broadcasted_iota(jnp.int32, (1, _P), 1) + base
    gx = (pidx & (_W - 1)).astype(jnp.float32)
    gy = (pidx >> 8).astype(jnp.float32)

    nt = pos_ref.shape[1]
    lane_io = jax.lax.broadcasted_iota(jnp.int32, (n, nt), 1)
    posc = jnp.sum(jnp.where(lane_io == i, pos_ref[...], 0),
                   axis=1, keepdims=True)

    iota_l = jax.lax.broadcasted_iota(jnp.int32, (n, n), 1)
    cmask = jnp.broadcast_to(posc, (n, n)) == iota_l
    cb16 = jnp.where(cmask, 1.0, 0.0).astype(jnp.bfloat16)
    comb = comb_ref[...]
    ch, cm, cl = _split3(comb)
    dn_c = (((0,), (0,)), ((), ()))
    compc = (jax.lax.dot_general(cb16, ch, dn_c,
                                 preferred_element_type=jnp.float32)
             + jax.lax.dot_general(cb16, cm, dn_c,
                                   preferred_element_type=jnp.float32)
             + jax.lax.dot_general(cb16, cl, dn_c,
                                   preferred_element_type=jnp.float32))

    ii = jax.lax.broadcasted_iota(jnp.int32, (_G, _G), 0)
    kk = jax.lax.broadcasted_iota(jnp.int32, (_G, _G), 1)
    tri = jnp.where(kk < ii, 1.0, 0.0).astype(jnp.bfloat16)

    acc_s[...] = jnp.zeros((8, _P), jnp.float32)
    tc_s[...] = jnp.ones((1, _P), jnp.float32)
    wmax_s[...] = jnp.zeros((1, _P), jnp.float32)
    obs_s[...] = jnp.zeros((1, _P), jnp.float32)
    dn_tri = (((1,), (0,)), ((), ()))
    dn_col = (((0,), (0,)), ((), ()))
    count = jnp.max(posc) + 1

    for cidx in range(n // _G):
        @pl.when(count > cidx * _G)
        def _():
            s = cidx * _G
            mx = compc[s:s + _G, 0:1]
            my = compc[s:s + _G, 1:2]
            ca = compc[s:s + _G, 2:3]
            cb = compc[s:s + _G, 3:4]
            cc = compc[s:s + _G, 4:5]
            opa = compc[s:s + _G, 5:6]

            dx = gx - mx
            dy = gy - my
            power = -0.5 * (ca * dx * dx + cc * dy * dy) - cb * dx * dy
            alpha = jnp.minimum(0.99, opa * jnp.exp(power))
            keep = (power <= 0.0) & (alpha >= _ALPHA_MIN)
            alpha = jnp.where(keep, alpha, 0.0)

            lg = jnp.log(1.0 - alpha)
            lg_h = lg.astype(jnp.bfloat16)
            lg_l = (lg - lg_h.astype(jnp.float32)).astype(jnp.bfloat16)
            sm = (jax.lax.dot_general(tri, lg_h, dn_tri,
                                      preferred_element_type=jnp.float32)
                  + jax.lax.dot_general(tri, lg_l, dn_tri,
                                        preferred_element_type=jnp.float32))
            tcv = tc_s[...]
            w = alpha * (tcv * jnp.exp(sm))

            bh = compc[s:s + _G, 8:16].astype(jnp.bfloat16)
            wh = w.astype(jnp.bfloat16)
            acc_s[...] = acc_s[...] + jax.lax.dot_general(
                bh, wh, dn_col, preferred_element_type=jnp.float32)
            obs_s[...] = obs_s[...] + jnp.sum(
                jnp.where(keep, 1.0, 0.0), axis=0, keepdims=True)
            wmax_s[...] = jnp.maximum(wmax_s[...],
                                      jnp.max(w, axis=0, keepdims=True))
            tc_s[...] = tcv * jnp.exp(sm[_G - 1:_G, :] + lg[_G - 1:_G, :])

    tc = tc_s[...]
    acc = acc_s[...]
    out = jnp.concatenate([
        acc[0:1, :] + tc * bg_ref[0],
        acc[1:2, :] + tc * bg_ref[1],
        acc[2:3, :] + tc * bg_ref[2],
        acc[3:4, :],
        wmax_s[...],
        1.0 - tc,
        acc[6:8, :],
    ], axis=0)
    pix_ref[...] = out
    obs_ref[...] = obs_s[...].astype(jnp.int32)


def kernel(means3D, opacities, colors_precomp, scales, rotations, bg, viewmatrix):
    n = means3D.shape[0]
    g = jnp.concatenate(
        [means3D.T, scales.T, rotations.T, opacities.T, colors_precomp.T],
        axis=0).astype(jnp.float32)
    vmf = viewmatrix.astype(jnp.float32)
    bgv = bg.astype(jnp.float32)
    nt_total = _H * _W // _P

    params, rad = pl.pallas_call(
        _prepass_body,
        out_shape=(jax.ShapeDtypeStruct((16, n), jnp.float32),
                   jax.ShapeDtypeStruct((1, n), jnp.int32)),
        in_specs=[pl.BlockSpec(memory_space=pltpu.VMEM),
                  pl.BlockSpec(memory_space=pltpu.SMEM)],
        out_specs=(pl.BlockSpec(memory_space=pltpu.VMEM),
                   pl.BlockSpec(memory_space=pltpu.VMEM)),
    )(g, vmf)

    comb, posm = pl.pallas_call(
        _sortbin_body,
        out_shape=(jax.ShapeDtypeStruct((n, 16), jnp.float32),
                   jax.ShapeDtypeStruct((n, nt_total), jnp.int32)),
        in_specs=[pl.BlockSpec(memory_space=pltpu.VMEM),
                  pl.BlockSpec(memory_space=pltpu.VMEM)],
        out_specs=(pl.BlockSpec(memory_space=pltpu.VMEM),
                   pl.BlockSpec(memory_space=pltpu.VMEM)),
    )(params, params.T)

    pix, obs = pl.pallas_call(
        _raster_body,
        grid=(nt_total,),
        in_specs=[pl.BlockSpec((n, 16), lambda i: (0, 0)),
                  pl.BlockSpec((n, nt_total), lambda i: (0, 0)),
                  pl.BlockSpec(memory_space=pltpu.SMEM)],
        out_specs=[pl.BlockSpec((8, _P), lambda i: (0, i)),
                   pl.BlockSpec((1, _P), lambda i: (0, i))],
        out_shape=[jax.ShapeDtypeStruct((8, _H * _W), jnp.float32),
                   jax.ShapeDtypeStruct((1, _H * _W), jnp.int32)],
        scratch_shapes=[pltpu.VMEM((8, _P), jnp.float32),
                        pltpu.VMEM((1, _P), jnp.float32),
                        pltpu.VMEM((1, _P), jnp.float32),
                        pltpu.VMEM((1, _P), jnp.float32)],
        compiler_params=pltpu.CompilerParams(
            dimension_semantics=("arbitrary",)),
    )(comb, posm, bgv)

    color = jnp.transpose(pix[0:3].reshape(3, _H, _W), (1, 2, 0))
    out_observe = obs.reshape(_H, _W)
    out_plane_depth = pix[3].reshape(_H, _W)
    app_opacity = pix[4].reshape(_H, _W)
    color_alpha = pix[5].reshape(_H, _W)
    return (color, rad[0], out_observe, out_plane_depth,
            app_opacity, color_alpha)


# speedup vs baseline: 1.1134x; 1.1134x over previous
import jax
import jax.numpy as jnp
from jax.experimental import pallas as pl
from jax.experimental.pallas import tpu as pltpu

_H = 256
_W = 256
_TANFOV = 0.7
_MIN_DEPTH = 0.2
_ALPHA_MIN = 1.0 / 255.0
_FX = _W / (2.0 * _TANFOV)
_FY = _H / (2.0 * _TANFOV)
_P = 4096
_G = 128


def _b(v):
    return v.astype(jnp.bfloat16).astype(jnp.float32)


def _prepass_body(g_ref, vm_ref, par_ref, rad_ref):
    m0 = g_ref[0:1, :]
    m1 = g_ref[1:2, :]
    m2 = g_ref[2:3, :]
    s0 = g_ref[3:4, :]
    s1 = g_ref[4:5, :]
    s2 = g_ref[5:6, :]
    qr = g_ref[6:7, :]
    qx = g_ref[7:8, :]
    qy = g_ref[8:9, :]
    qz = g_ref[9:10, :]
    opa = g_ref[10:11, :]

    vm = [[vm_ref[j, k] for k in range(4)] for j in range(4)]
    rb = [[_b(vm[j][k]) for k in range(3)] for j in range(3)]
    mb0, mb1, mb2 = _b(m0), _b(m1), _b(m2)
    xv = ((mb0 * rb[0][0] + mb1 * rb[0][1]) + mb2 * rb[0][2]) + vm[0][3]
    yv = ((mb0 * rb[1][0] + mb1 * rb[1][1]) + mb2 * rb[1][2]) + vm[1][3]
    zv = ((mb0 * rb[2][0] + mb1 * rb[2][1]) + mb2 * rb[2][2]) + vm[2][3]

    zc = jnp.maximum(zv, 1e-4)
    lim_x = 1.3 * _TANFOV
    lim_y = 1.3 * _TANFOV
    txc = jnp.clip(xv / zc, -lim_x, lim_x) * zc
    tyc = jnp.clip(yv / zc, -lim_y, lim_y) * zc

    qnorm = jnp.sqrt(qr * qr + qx * qx + qy * qy + qz * qz)
    r_ = qr / qnorm
    x_ = qx / qnorm
    y_ = qy / qnorm
    z_ = qz / qnorm
    r00 = 1.0 - 2.0 * (y_ * y_ + z_ * z_)
    r01 = 2.0 * (x_ * y_ - r_ * z_)
    r02 = 2.0 * (x_ * z_ + r_ * y_)
    r10 = 2.0 * (x_ * y_ + r_ * z_)
    r11 = 1.0 - 2.0 * (x_ * x_ + z_ * z_)
    r12 = 2.0 * (y_ * z_ - r_ * x_)
    r20 = 2.0 * (x_ * z_ - r_ * y_)
    r21 = 2.0 * (y_ * z_ + r_ * x_)
    r22 = 1.0 - 2.0 * (x_ * x_ + y_ * y_)

    mb = [[_b(r00 * s0), _b(r01 * s1), _b(r02 * s2)],
          [_b(r10 * s0), _b(r11 * s1), _b(r12 * s2)],
          [_b(r20 * s0), _b(r21 * s1), _b(r22 * s2)]]
    sg = [[None] * 3 for _ in range(3)]
    for si in range(3):
        for sj in range(si, 3):
            sg[si][sj] = ((mb[si][0] * mb[sj][0] + mb[si][1] * mb[sj][1])
                          + mb[si][2] * mb[sj][2])
            sg[sj][si] = sg[si][sj]

    j00 = _b(_FX / zc)
    j02 = _b(-_FX * txc / (zc * zc))
    j11 = _b(_FY / zc)
    j12 = _b(-_FY * tyc / (zc * zc))
    t = [[j00 * rb[0][k] + j02 * rb[2][k] for k in range(3)],
         [j11 * rb[1][k] + j12 * rb[2][k] for k in range(3)]]

    tb = [[_b(t[ti][k]) for k in range(3)] for ti in range(2)]
    sgb = [[_b(sg[si][sj]) for sj in range(3)] for si in range(3)]
    u = [[((tb[ti][0] * sgb[0][k] + tb[ti][1] * sgb[1][k])
           + tb[ti][2] * sgb[2][k]) for k in range(3)] for ti in range(2)]
    ub = [[_b(u[ti][k]) for k in range(3)] for ti in range(2)]
    a = ((ub[0][0] * tb[0][0] + ub[0][1] * tb[0][1]) + ub[0][2] * tb[0][2]) + 0.3
    b = ((ub[0][0] * tb[1][0] + ub[0][1] * tb[1][1]) + ub[0][2] * tb[1][2])
    c = ((ub[1][0] * tb[1][0] + ub[1][1] * tb[1][1]) + ub[1][2] * tb[1][2]) + 0.3

    det = a * c - b * b
    inv_det = 1.0 / jnp.maximum(det, 1e-12)
    ca = c * inv_det
    cb = -b * inv_det
    cc = a * inv_det
    mid = 0.5 * (a + c)
    lam1 = mid + jnp.sqrt(jnp.maximum(mid * mid - det, 0.1))
    radii = jnp.ceil(3.0 * jnp.sqrt(lam1)).astype(jnp.int32)
    mx = _FX * xv / zc + 0.5 * _W - 0.5
    my = _FY * yv / zc + 0.5 * _H - 0.5
    validf = jnp.where((zv > _MIN_DEPTH) & (det > 0.0), 1.0, 0.0)

    opa_eff = opa * validf

    amin_s = _ALPHA_MIN * (1.0 - 1e-6)
    qmax = 2.0 * jnp.log(jnp.maximum(opa_eff, 1e-30) / amin_s)
    ry = jnp.where(qmax > 0.0,
                   jnp.sqrt(jnp.maximum(qmax, 0.0) * c) * (1.0 + 1e-5) + 0.01,
                   -1e9)
    rx = jnp.sqrt(jnp.maximum(qmax, 0.0) * a) * (1.0 + 1e-5) + 0.01
    x_on = (mx + rx >= -0.01) & (mx - rx <= (_W - 1) + 0.01)
    reach = jnp.where(x_on, ry, -1e9)

    par_ref[0:1, :] = mx
    par_ref[1:2, :] = my
    par_ref[2:3, :] = ca
    par_ref[3:4, :] = cb
    par_ref[4:5, :] = cc
    par_ref[5:6, :] = opa_eff
    par_ref[6:7, :] = reach
    par_ref[7:8, :] = zv
    par_ref[8:11, :] = g_ref[11:14, :]
    par_ref[11:12, :] = zv
    par_ref[12:16, :] = jnp.zeros((4, zv.shape[1]), jnp.float32)
    rad_ref[...] = radii


def _split3(v):
    vh = v.astype(jnp.bfloat16)
    v1 = v - vh.astype(jnp.float32)
    vm = v1.astype(jnp.bfloat16)
    vl = (v1 - vm.astype(jnp.float32)).astype(jnp.bfloat16)
    return vh, vm, vl


def _sortbin_body(par_ref, parT_ref, comb_ref, posm_ref):
    n = par_ref.shape[1]
    nt = posm_ref.shape[1]
    rpt = _P // _W
    f32 = jnp.float32
    io0 = jax.lax.broadcasted_iota(jnp.int32, (n, n), 0)
    io1 = jax.lax.broadcasted_iota(jnp.int32, (n, n), 1)
    dn_row = (((1,), (0,)), ((), ()))

    zr = par_ref[7:8, :]
    zc = parT_ref[:, 7:8]
    mm = (zc < zr) | ((zc == zr) & (io0 < io1))
    mf = jnp.where(mm, 1.0, 0.0).astype(jnp.bfloat16)
    ones_row = jnp.ones((1, n), jnp.bfloat16)
    rank = jax.lax.dot_general(ones_row, mf, dn_row,
                               preferred_element_type=f32)
    ranki = rank.astype(jnp.int32)
    psort = jnp.where(jnp.broadcast_to(ranki, (n, n)) == io0,
                      1.0, 0.0).astype(jnp.bfloat16)

    parT = parT_ref[...]
    ph, pm, plo = _split3(parT)
    comb_s = (jax.lax.dot_general(psort, ph, dn_row,
                                  preferred_element_type=f32)
              + jax.lax.dot_general(psort, pm, dn_row,
                                    preferred_element_type=f32)
              + jax.lax.dot_general(psort, plo, dn_row,
                                    preferred_element_type=f32))
    comb_ref[...] = comb_s

    my_sc = comb_s[:, 1:2]
    re_sc = comb_s[:, 6:7]
    y0 = (jax.lax.broadcasted_iota(jnp.int32, (n, nt), 1)
          * rpt).astype(f32)
    myb = jnp.broadcast_to(my_sc, (n, nt))
    reb = jnp.broadcast_to(re_sc, (n, nt))
    touch = (myb + reb >= y0) & (myb - reb <= y0 + (rpt - 1))
    touch16 = jnp.where(touch, 1.0, 0.0).astype(jnp.bfloat16)
    low_incl = jnp.where(io1 <= io0, 1.0, 0.0).astype(jnp.bfloat16)
    pos = jax.lax.dot_general(low_incl, touch16, dn_row,
                              preferred_element_type=f32)
    posm_ref[...] = jnp.where(touch, pos - 1.0, -1.0).astype(jnp.int32)


def _raster_body(comb_ref, pos_ref, bg_ref,
                 pix_ref, obs_ref, acc_s, tc_s, wmax_s, obs_s):
    i = pl.program_id(0)
    base = i * _P
    n = comb_ref.shape[0]
    pidx = jax.lax.broadcasted_iota(jnp.int32, (1, _P), 1) + base
    gx = (pidx & (_W - 1)).astype(jnp.float32)
    gy = (pidx >> 8).astype(jnp.float32)

    nt = pos_ref.shape[1]
    lane_io = jax.lax.broadcasted_iota(jnp.int32, (n, nt), 1)
    posc = jnp.sum(jnp.where(lane_io == i, pos_ref[...], 0),
                   axis=1, keepdims=True)

    iota_l = jax.lax.broadcasted_iota(jnp.int32, (n, n), 1)
    cmask = jnp.broadcast_to(posc, (n, n)) == iota_l
    cb16 = jnp.where(cmask, 1.0, 0.0).astype(jnp.bfloat16)
    comb = comb_ref[...]
    ch, cm, cl = _split3(comb)
    dn_c = (((0,), (0,)), ((), ()))
    compc = (jax.lax.dot_general(cb16, ch, dn_c,
                                 preferred_element_type=jnp.float32)
             + jax.lax.dot_general(cb16, cm, dn_c,
                                   preferred_element_type=jnp.float32)
             + jax.lax.dot_general(cb16, cl, dn_c,
                                   preferred_element_type=jnp.float32))

    ii = jax.lax.broadcasted_iota(jnp.int32, (_G, _G), 0)
    kk = jax.lax.broadcasted_iota(jnp.int32, (_G, _G), 1)
    tri = jnp.where(kk < ii, 1.0, 0.0).astype(jnp.bfloat16)

    acc_s[...] = jnp.zeros((8, _P), jnp.float32)
    tc_s[...] = jnp.ones((1, _P), jnp.float32)
    wmax_s[...] = jnp.zeros((1, _P), jnp.float32)
    obs_s[...] = jnp.zeros((1, _P), jnp.float32)
    dn_tri = (((1,), (0,)), ((), ()))
    dn_col = (((0,), (0,)), ((), ()))
    count = jnp.max(posc) + 1

    for cidx in range(n // _G):
        @pl.when(count > cidx * _G)
        def _():
            s = cidx * _G
            mx = compc[s:s + _G, 0:1]
            my = compc[s:s + _G, 1:2]
            ca = compc[s:s + _G, 2:3]
            cb = compc[s:s + _G, 3:4]
            cc = compc[s:s + _G, 4:5]
            opa = compc[s:s + _G, 5:6]

            dx = gx - mx
            dy = gy - my
            power = -0.5 * (ca * dx * dx + cc * dy * dy) - cb * dx * dy
            alpha = jnp.minimum(0.99, opa * jnp.exp(power))
            keep = (power <= 0.0) & (alpha >= _ALPHA_MIN)
            alpha = jnp.where(keep, alpha, 0.0)

            lg = jnp.log(1.0 - alpha)
            lg_h = lg.astype(jnp.bfloat16)
            lg_l = (lg - lg_h.astype(jnp.float32)).astype(jnp.bfloat16)
            sm = (jax.lax.dot_general(tri, lg_h, dn_tri,
                                      preferred_element_type=jnp.float32)
                  + jax.lax.dot_general(tri, lg_l, dn_tri,
                                        preferred_element_type=jnp.float32))
            tcv = tc_s[...]
            w = alpha * (tcv * jnp.exp(sm))

            bh = compc[s:s + _G, 8:16].astype(jnp.bfloat16)
            wh = w.astype(jnp.bfloat16)
            acc_s[...] = acc_s[...] + jax.lax.dot_general(
                bh, wh, dn_col, preferred_element_type=jnp.float32)
            obs_s[...] = obs_s[...] + jnp.sum(
                jnp.where(keep, 1.0, 0.0), axis=0, keepdims=True)
            wmax_s[...] = jnp.maximum(wmax_s[...],
                                      jnp.max(w, axis=0, keepdims=True))
            tc_s[...] = tcv * jnp.exp(sm[_G - 1:_G, :] + lg[_G - 1:_G, :])

    tc = tc_s[...]
    acc = acc_s[...]
    out = jnp.concatenate([
        acc[0:1, :] + tc * bg_ref[0],
        acc[1:2, :] + tc * bg_ref[1],
        acc[2:3, :] + tc * bg_ref[2],
        acc[3:4, :],
        wmax_s[...],
        1.0 - tc,
        acc[6:8, :],
    ], axis=0)
    pix_ref[...] = out
    obs_ref[...] = obs_s[...].astype(jnp.int32)


def kernel(means3D, opacities, colors_precomp, scales, rotations, bg, viewmatrix):
    n = means3D.shape[0]
    g = jnp.concatenate(
        [means3D.T, scales.T, rotations.T, opacities.T, colors_precomp.T],
        axis=0).astype(jnp.float32)
    vmf = viewmatrix.astype(jnp.float32)
    bgv = bg.astype(jnp.float32)
    nt_total = _H * _W // _P

    params, rad = pl.pallas_call(
        _prepass_body,
        out_shape=(jax.ShapeDtypeStruct((16, n), jnp.float32),
                   jax.ShapeDtypeStruct((1, n), jnp.int32)),
        in_specs=[pl.BlockSpec(memory_space=pltpu.VMEM),
                  pl.BlockSpec(memory_space=pltpu.SMEM)],
        out_specs=(pl.BlockSpec(memory_space=pltpu.VMEM),
                   pl.BlockSpec(memory_space=pltpu.VMEM)),
    )(g, vmf)

    comb, posm = pl.pallas_call(
        _sortbin_body,
        out_shape=(jax.ShapeDtypeStruct((n, 16), jnp.float32),
                   jax.ShapeDtypeStruct((n, nt_total), jnp.int32)),
        in_specs=[pl.BlockSpec(memory_space=pltpu.VMEM),
                  pl.BlockSpec(memory_space=pltpu.VMEM)],
        out_specs=(pl.BlockSpec(memory_space=pltpu.VMEM),
                   pl.BlockSpec(memory_space=pltpu.VMEM)),
    )(params, params.T)

    pix, obs = pl.pallas_call(
        _raster_body,
        grid=(nt_total,),
        in_specs=[pl.BlockSpec((n, 16), lambda i: (0, 0)),
                  pl.BlockSpec((n, nt_total), lambda i: (0, 0)),
                  pl.BlockSpec(memory_space=pltpu.SMEM)],
        out_specs=[pl.BlockSpec((8, _P), lambda i: (0, i)),
                   pl.BlockSpec((1, _P), lambda i: (0, i))],
        out_shape=[jax.ShapeDtypeStruct((8, _H * _W), jnp.float32),
                   jax.ShapeDtypeStruct((1, _H * _W), jnp.int32)],
        scratch_shapes=[pltpu.VMEM((8, _P), jnp.float32),
                        pltpu.VMEM((1, _P), jnp.float32),
                        pltpu.VMEM((1, _P), jnp.float32),
                        pltpu.VMEM((1, _P), jnp.float32)],
        compiler_params=pltpu.CompilerParams(
            dimension_semantics=("arbitrary",)),
    )(comb, posm, bgv)

    color = jnp.transpose(pix[0:3].reshape(3, _H, _W), (1, 2, 0))
    out_observe = obs.reshape(_H, _W)
    out_plane_depth = pix[3].reshape(_H, _W)
    app_opacity = pix[4].reshape(_H, _W)
    color_alpha = pix[5].reshape(_H, _W)
    return (color, rad[0], out_observe, out_plane_depth,
            app_opacity, color_alpha)
